# baseline (speedup 1.0000x reference)
import functools

import jax
import jax.numpy as jnp
from jax import lax
from jax.experimental import pallas as pl
from jax.experimental.pallas import tpu as pltpu
from jax.experimental.pallas import tpu_sc as plsc

N = 50000
E = 800000
D_NODE = 16
D_REV = 64

NC = 2
NS = 16
NW = NC * NS

NPAD = 50048

ET = E // 128
TPW = ET // NW
EXTRA = ET - NW * TPW
KT = 13
CHUNK = KT * 128
NCHUNK = TPW // KT
ROWS_PER_TILE = NPAD // NS

TBLK = 50


def _h_body(xt_ref, w_ref, o_ref):
    o_ref[...] = lax.dot_general(xt_ref[...], w_ref[...], (((0,), (0,)), ((), ())),
                                 preferred_element_type=jnp.float32)


def _rf_body(rft_ref, w_ref, o_ref):
    y = lax.dot_general(w_ref[...], rft_ref[...], (((0,), (0,)), ((), ())),
                        preferred_element_type=jnp.float32)
    o_ref[...] = jnp.transpose(y.reshape(D_NODE, TBLK, 128), (1, 0, 2))


def _add_body(a_ref, b_ref, o_ref):
    o_ref[...] = a_ref[...] + b_ref[...]


_sc_mesh = plsc.VectorSubcoreMesh(core_axis_name="c", subcore_axis_name="s")


@functools.partial(
    pl.kernel,
    out_type=jax.ShapeDtypeStruct((NC, NPAD, D_NODE), jnp.float32),
    mesh=_sc_mesh,
    scratch_types=[
        pltpu.VMEM((CHUNK,), jnp.int32),
        pltpu.VMEM((CHUNK,), jnp.int32),
        pltpu.VMEM((CHUNK,), jnp.float32),
        pltpu.VMEM((CHUNK, D_NODE), jnp.float32),
        pltpu.VMEM((KT, D_NODE, 128), jnp.float32),
        pltpu.VMEM((128,), jnp.int32),
        pltpu.VMEM((128,), jnp.int32),
        pltpu.VMEM((128,), jnp.float32),
        pltpu.VMEM((128, D_NODE), jnp.float32),
        pltpu.VMEM((1, D_NODE, 128), jnp.float32),
        pltpu.VMEM_SHARED((NPAD, D_NODE), jnp.float32),
        pltpu.SemaphoreType.DMA,
    ],
    compiler_params=pltpu.CompilerParams(use_tc_tiling_on_sc=False,
                                         needs_layout_passes=False),
)
def _sc_scatter(h_hbm, rf_hbm, src_hbm, dst_hbm, w_hbm, zeros_hbm, out_hbm,
                sidx_v, didx_v, w_v, g_v, rf_v,
                sidx_t, didx_t, w_t, g_t, rf_t, acc, sem):

    lane16 = lax.iota(jnp.int32, 16)

    def process(t0, kt, sidx, didx, wv, gv, rfv):
        e0 = t0 * 128
        n = kt * 128
        pltpu.sync_copy(src_hbm.at[pl.ds(e0, n)], sidx)
        pltpu.sync_copy(dst_hbm.at[pl.ds(e0, n)], didx)
        pltpu.sync_copy(w_hbm.at[pl.ds(e0, n)], wv)
        pltpu.sync_copy(rf_hbm.at[pl.ds(t0, kt)], rfv)
        pltpu.async_copy(h_hbm.at[sidx], gv, sem).wait()

        def body(k, _):
            i0 = k * 16
            wvec = wv[pl.ds(i0, 16)]
            tt = i0 // 128
            l0 = lax.rem(i0, 128)
            rows = i0 + lane16
            cols = [jnp.full((16,), d, jnp.int32) for d in range(16)]
            gcols = [plsc.load_gather(gv, [rows, cols[d]]) for d in range(16)]
            ms = [(gcols[d] + rfv[tt, d, pl.ds(l0, 16)]) * wvec
                  for d in range(16)]
            for d in range(16):
                plsc.store_scatter(gv, [rows, cols[d]], ms[d])
            return 0

        lax.fori_loop(0, n // 16, body, 0)
        pltpu.sync_copy(gv, acc.at[didx], add=True)

    cid = lax.axis_index("c")
    sid = lax.axis_index("s")
    wid = cid * NS + sid

    pltpu.sync_copy(zeros_hbm, acc.at[pl.ds(sid * ROWS_PER_TILE, ROWS_PER_TILE)])
    plsc.subcore_barrier()

    base_t = wid * TPW

    for g in range(NCHUNK):
        process(base_t + g * KT, KT, sidx_v, didx_v, w_v, g_v, rf_v)

    @pl.when(wid < EXTRA)
    def _():
        process(NW * TPW + wid, 1, sidx_t, didx_t, w_t, g_t, rf_t)

    plsc.subcore_barrier()
    pltpu.sync_copy(acc.at[pl.ds(sid * ROWS_PER_TILE, ROWS_PER_TILE)],
                    out_hbm.at[cid, pl.ds(sid * ROWS_PER_TILE, ROWS_PER_TILE)])


def kernel(x, edge_index, review_feat, edge_w, node_W, review_W):
    zeros = jnp.zeros((ROWS_PER_TILE, D_NODE), jnp.float32)
    src = edge_index[0]
    dst = edge_index[1]
    w_flat = edge_w.reshape(E)
    xt = x.T
    rft = review_feat.T

    h = pl.pallas_call(
        _h_body,
        out_shape=jax.ShapeDtypeStruct((N, D_NODE), jnp.float32),
    )(xt, node_W)

    rf = pl.pallas_call(
        _rf_body,
        grid=(ET // TBLK,),
        in_specs=[
            pl.BlockSpec((D_REV, TBLK * 128), lambda i: (0, i)),
            pl.BlockSpec((D_REV, D_NODE), lambda i: (0, 0)),
        ],
        out_specs=pl.BlockSpec((TBLK, D_NODE, 128), lambda i: (i, 0, 0)),
        out_shape=jax.ShapeDtypeStruct((ET, D_NODE, 128), jnp.float32),
    )(rft, review_W)

    partials = _sc_scatter(h, rf, src, dst, w_flat, zeros)

    FLAT = NPAD * D_NODE
    CB = 1024 * 34
    p0 = partials[0].reshape(FLAT)
    p1 = partials[1].reshape(FLAT)
    sum1d = pl.pallas_call(
        _add_body,
        grid=(FLAT // CB,),
        in_specs=[
            pl.BlockSpec((CB,), lambda i: (i,)),
            pl.BlockSpec((CB,), lambda i: (i,)),
        ],
        out_specs=pl.BlockSpec((CB,), lambda i: (i,)),
        out_shape=jax.ShapeDtypeStruct((FLAT,), jnp.float32),
    )(p0, p1)
    return sum1d[:N * D_NODE].reshape(N, D_NODE)


# speedup vs baseline: 1.0714x; 1.0714x over previous
import functools

import jax
import jax.numpy as jnp
from jax import lax
from jax.experimental import pallas as pl
from jax.experimental.pallas import tpu as pltpu
from jax.experimental.pallas import tpu_sc as plsc

N = 50000
E = 800000
D_NODE = 16
D_REV = 64

NC = 2
NS = 16
NW = NC * NS

NPAD = 50048

ET = E // 128
TPW = ET // NW
EXTRA = ET - NW * TPW
KT = 13
CHUNK = KT * 128
NCHUNK = TPW // KT
ROWS_PER_TILE = NPAD // NS

TBLK = 50


def _h_body(xt_ref, w_ref, o_ref):
    o_ref[...] = lax.dot_general(xt_ref[...], w_ref[...], (((0,), (0,)), ((), ())),
                                 preferred_element_type=jnp.float32)


def _rf_body(rft_ref, w_ref, o_ref):
    y = lax.dot_general(w_ref[...], rft_ref[...], (((0,), (0,)), ((), ())),
                        preferred_element_type=jnp.float32)
    o_ref[...] = jnp.transpose(y.reshape(D_NODE, TBLK, 128), (1, 0, 2))


def _add_body(a_ref, b_ref, o_ref):
    o_ref[...] = a_ref[...] + b_ref[...]


_sc_mesh = plsc.VectorSubcoreMesh(core_axis_name="c", subcore_axis_name="s")


@functools.partial(
    pl.kernel,
    out_type=jax.ShapeDtypeStruct((NC, NPAD, D_NODE), jnp.float32),
    mesh=_sc_mesh,
    scratch_types=[
        pltpu.VMEM((CHUNK,), jnp.int32),
        pltpu.VMEM((CHUNK,), jnp.int32),
        pltpu.VMEM((CHUNK,), jnp.float32),
        pltpu.VMEM((CHUNK, D_NODE), jnp.float32),
        pltpu.VMEM((KT, D_NODE, 128), jnp.float32),
        pltpu.VMEM((128,), jnp.int32),
        pltpu.VMEM((128,), jnp.int32),
        pltpu.VMEM((128,), jnp.float32),
        pltpu.VMEM((128, D_NODE), jnp.float32),
        pltpu.VMEM((1, D_NODE, 128), jnp.float32),
        pltpu.VMEM_SHARED((NPAD, D_NODE), jnp.float32),
        pltpu.SemaphoreType.DMA,
    ],
    compiler_params=pltpu.CompilerParams(use_tc_tiling_on_sc=False,
                                         needs_layout_passes=False),
)
def _sc_scatter(h_hbm, rf_hbm, src_hbm, dst_hbm, w_hbm, zeros_hbm, out_hbm,
                sidx_v, didx_v, w_v, g_v, rf_v,
                sidx_t, didx_t, w_t, g_t, rf_t, acc, sem):

    lane16 = lax.iota(jnp.int32, 16)

    def process(t0, kt, sidx, didx, wv, gv, rfv):
        e0 = t0 * 128
        n = kt * 128
        pltpu.sync_copy(src_hbm.at[pl.ds(e0, n)], sidx)
        pltpu.sync_copy(dst_hbm.at[pl.ds(e0, n)], didx)
        pltpu.sync_copy(w_hbm.at[pl.ds(e0, n)], wv)
        pltpu.sync_copy(rf_hbm.at[pl.ds(t0, kt)], rfv)
        pltpu.async_copy(h_hbm.at[sidx], gv, sem).wait()

        def body(k, _):
            i0 = k * 16
            wvec = wv[pl.ds(i0, 16)]
            tt = i0 // 128
            l0 = lax.rem(i0, 128)
            rows = i0 + lane16
            cols = [jnp.full((16,), d, jnp.int32) for d in range(16)]
            gcols = [plsc.load_gather(gv, [rows, cols[d]]) for d in range(16)]
            ms = [(gcols[d] + rfv[tt, d, pl.ds(l0, 16)]) * wvec
                  for d in range(16)]
            for d in range(16):
                plsc.store_scatter(gv, [rows, cols[d]], ms[d])
            return 0

        lax.fori_loop(0, n // 16, body, 0)
        pltpu.sync_copy(gv, acc.at[didx], add=True)

    cid = lax.axis_index("c")
    sid = lax.axis_index("s")
    wid = cid * NS + sid

    pltpu.sync_copy(zeros_hbm, acc.at[pl.ds(sid * ROWS_PER_TILE, ROWS_PER_TILE)])
    plsc.subcore_barrier()

    base_t = wid * TPW

    for g in range(NCHUNK):
        process(base_t + g * KT, KT, sidx_v, didx_v, w_v, g_v, rf_v)

    @pl.when(wid < EXTRA)
    def _():
        process(NW * TPW + wid, 1, sidx_t, didx_t, w_t, g_t, rf_t)

    plsc.subcore_barrier()
    pltpu.sync_copy(acc.at[pl.ds(sid * ROWS_PER_TILE, ROWS_PER_TILE)],
                    out_hbm.at[cid, pl.ds(sid * ROWS_PER_TILE, ROWS_PER_TILE)])


def kernel(x, edge_index, review_feat, edge_w, node_W, review_W):
    zeros = jnp.zeros((ROWS_PER_TILE, D_NODE), jnp.float32)
    src = edge_index[0]
    dst = edge_index[1]
    w_flat = edge_w.reshape(E)
    xt = x.T
    rft = review_feat.T

    h = pl.pallas_call(
        _h_body,
        out_shape=jax.ShapeDtypeStruct((N, D_NODE), jnp.float32),
    )(xt, node_W)

    rf = pl.pallas_call(
        _rf_body,
        grid=(ET // TBLK,),
        in_specs=[
            pl.BlockSpec((D_REV, TBLK * 128), lambda i: (0, i)),
            pl.BlockSpec((D_REV, D_NODE), lambda i: (0, 0)),
        ],
        out_specs=pl.BlockSpec((TBLK, D_NODE, 128), lambda i: (i, 0, 0)),
        out_shape=jax.ShapeDtypeStruct((ET, D_NODE, 128), jnp.float32),
    )(rft, review_W)

    partials = _sc_scatter(h, rf, src, dst, w_flat, zeros)

    rst = pl.pallas_call(
        _add_body,
        grid=(10,),
        in_specs=[
            pl.BlockSpec((N // 10, D_NODE), lambda i: (i, 0)),
            pl.BlockSpec((N // 10, D_NODE), lambda i: (i, 0)),
        ],
        out_specs=pl.BlockSpec((N // 10, D_NODE), lambda i: (i, 0)),
        out_shape=jax.ShapeDtypeStruct((N, D_NODE), jnp.float32),
    )(partials[0], partials[1])
    return rst
